# baseline (speedup 1.0000x reference)
import functools

import jax
import jax.numpy as jnp
from jax import lax
from jax.experimental import pallas as pl
from jax.experimental.pallas import tpu as pltpu
from jax.experimental.pallas import tpu_sc as plsc

N_USERS = 20000
N_ITEMS = 30000
NN = N_USERS + N_ITEMS
NPAD = 50048
E = 800000
DH = 32
BATCH = 16384
NC, NS = 2, 16
EBLK = E // 128
RB = 64
RBLK = NPAD // RB
DBLK = NPAD // 128
BPT = BATCH // NS

_MAGIC = 0x5F3759DF


def _vrsqrt(x):
    i = lax.bitcast_convert_type(x, jnp.int32)
    y = lax.bitcast_convert_type(jnp.int32(_MAGIC) - (i >> 1), jnp.float32)
    for _ in range(3):
        y = y * (1.5 - 0.5 * x * y * y)
    return y


def _body(users, items, e0s, packed,
          partials, ubuf,
          acc_sh, dib_sh, dsb_sh,
          pidx, rows, z1, ones1, dchunk, dibv, dsbv,
          stg, aux, cch, gout,
          isem, gsem, ssem, uwsem, esem):
    c = lax.axis_index("c")
    s = lax.axis_index("s")
    coff = c * NPAD
    ceb = c * EBLK
    eb0 = (EBLK * s) // NS
    eb1 = (EBLK * (s + 1)) // NS
    rb0 = (RBLK * s) // NS
    rb1 = (RBLK * (s + 1)) // NS
    db0 = (DBLK * s) // NS
    db1 = (DBLK * (s + 1)) // NS
    iota16 = lax.iota(jnp.int32, 16)
    zv = jnp.zeros((16,), jnp.float32)
    ov = jnp.ones((16,), jnp.float32)
    ns = jax.named_scope

    for g in range(8):
        z1[pl.ds(g * 16, 16)] = zv
        ones1[pl.ds(g * 16, 16)] = ov
    for r in range(RB):
        stg[0, r, pl.ds(0, 16)] = zv
        stg[0, r, pl.ds(16, 16)] = zv

    def za_blk(i, carry):
        pltpu.sync_copy(stg.at[0], acc_sh.at[pl.ds(i * RB, RB), :])
        return carry

    def zd_blk(i, carry):
        pltpu.sync_copy(z1, dib_sh.at[pl.ds(i * 128, 128)])
        return carry
    with ns("p1_zero"):
        lax.fori_loop(rb0, rb1, za_blk, 0)
        lax.fori_loop(db0, db1, zd_blk, 0)
    plsc.subcore_barrier()

    def dfire(i):
        q = i & 3
        pltpu.async_copy(packed.at[ceb + i], pidx.at[pl.ds(2 * q, 2), :],
                         isem.at[q])

    def dwait(i):
        q = i & 3
        pltpu.make_async_copy(packed.at[ceb + i], pidx.at[pl.ds(2 * q, 2), :],
                              isem.at[q]).wait()

    def dscat_wait(i):
        pltpu.make_async_copy(ones1, dib_sh.at[pidx.at[2 * (i & 3) + 1]],
                              ssem.at[i & 1]).wait()

    dfire(eb0)

    @pl.when(eb0 + 1 < eb1)
    def _d1():
        dfire(eb0 + 1)

    def deg_blk(i, carry):
        @pl.when(i - 2 >= eb0)
        def _w():
            dscat_wait(i - 2)

        @pl.when(i + 2 < eb1)
        def _f():
            dfire(i + 2)
        dwait(i)
        pltpu.async_copy(ones1, dib_sh.at[pidx.at[2 * (i & 3) + 1]], ssem.at[i & 1], add=True)
        return carry
    with ns("p2_deg"):
        lax.fori_loop(eb0, eb1, deg_blk, 0)

    @pl.when(eb1 - 2 >= eb0)
    def _dw2():
        dscat_wait(eb1 - 2)
    dscat_wait(eb1 - 1)
    plsc.subcore_barrier()

    def e0_fire(i, slot):
        pltpu.async_copy(e0s.at[pl.ds(coff + i * RB, RB), :], aux.at[slot],
                         esem.at[slot])

    def e0_wait(i, slot):
        pltpu.make_async_copy(e0s.at[pl.ds(coff + i * RB, RB), :], aux.at[slot],
                              esem.at[slot]).wait()

    def u_fire(i, slot):
        pltpu.async_copy(stg.at[slot], ubuf.at[pl.ds(coff + i * RB, RB), :],
                         uwsem.at[slot])

    def u_wait(i, slot):
        pltpu.make_async_copy(stg.at[slot], ubuf.at[pl.ds(coff + i * RB, RB), :],
                              uwsem.at[slot]).wait()

    e0_fire(rb0, 0)

    def prep_blk(i, carry):
        p = (i - rb0) & 1

        @pl.when(i - 2 >= rb0)
        def _uw():
            u_wait(i - 2, p)

        @pl.when(i + 1 < rb1)
        def _ef():
            e0_fire(i + 1, 1 - p)
        base = i * RB
        pltpu.sync_copy(dib_sh.at[pl.ds(base, RB)], dchunk)
        for g in range(RB // 16):
            dv = dchunk[pl.ds(g * 16, 16)] + 1.0
            dibv[pl.ds(g * 16, 16)] = 1.0 / dv
            dsbv[pl.ds(g * 16, 16)] = _vrsqrt(dv)
        pltpu.sync_copy(dibv, dib_sh.at[pl.ds(base, RB)])
        pltpu.sync_copy(dsbv, dsb_sh.at[pl.ds(base, RB)])
        e0_wait(i, p)

        def rowfn(r, carry2):
            bv = plsc.load_gather(dsbv, [jnp.full((16,), r, jnp.int32)])
            for h in (0, 16):
                stg[p, r, pl.ds(h, 16)] = bv * aux[p, r, pl.ds(h, 16)]
            return carry2
        lax.fori_loop(0, RB, rowfn, 0)
        u_fire(i, p)
        return carry
    with ns("p3_prep"):
        lax.fori_loop(rb0, rb1, prep_blk, 0)
    u_wait(rb1 - 2, (rb1 - 2 - rb0) & 1)
    u_wait(rb1 - 1, (rb1 - 1 - rb0) & 1)
    plsc.subcore_barrier()

    idx_fire = dfire
    idx_wait = dwait

    def gather_fire(i):
        pltpu.async_copy(ubuf.at[pidx.at[2 * (i & 3)]], rows.at[i % 3],
                         gsem.at[i % 3])

    def gather_wait(i):
        pltpu.make_async_copy(ubuf.at[pidx.at[2 * (i & 3)]], rows.at[i % 3],
                              gsem.at[i % 3]).wait()

    def ascat_wait(i):
        pltpu.make_async_copy(rows.at[i % 3], acc_sh.at[pidx.at[2 * (i & 3) + 1]],
                              ssem.at[i & 1]).wait()

    def layer_scatter():
        idx_fire(eb0)
        idx_fire(eb0 + 1)
        idx_fire(eb0 + 2)
        idx_wait(eb0)
        gather_fire(eb0)
        idx_wait(eb0 + 1)
        gather_fire(eb0 + 1)

        def eblk(i, carry):
            @pl.when(i - 1 >= eb0)
            def _sw():
                ascat_wait(i - 1)

            @pl.when(i + 3 < eb1)
            def _if():
                idx_fire(i + 3)

            @pl.when(i + 2 < eb1)
            def _gf():
                idx_wait(i + 2)
                gather_fire(i + 2)
            gather_wait(i)
            pltpu.async_copy(rows.at[i % 3], acc_sh.at[pidx.at[2 * (i & 3) + 1]],
                             ssem.at[i & 1], add=True)
            return carry
        with ns("pB_scatter"):
            lax.fori_loop(eb0, eb1, eblk, 0)
        ascat_wait(eb1 - 1)

    def a_fire(i, slot, k):
        if k == 2:
            pltpu.async_copy(ubuf.at[pl.ds(coff + i * RB, RB), :], aux.at[slot],
                             esem.at[slot])
        else:
            e0_fire(i, slot)

    def a_wait(i, slot, k):
        if k == 2:
            pltpu.make_async_copy(ubuf.at[pl.ds(coff + i * RB, RB), :],
                                  aux.at[slot], esem.at[slot]).wait()
        else:
            e0_wait(i, slot)

    def phase_c(k):
        if k != 1:
            a_fire(rb0, 0, k)

        def nblkfn(i, carry):
            p = (i - rb0) & 1

            @pl.when(i - 2 >= rb0)
            def _uw():
                u_wait(i - 2, p)

            if k != 1:
                @pl.when(i + 1 < rb1)
                def _af():
                    a_fire(i + 1, 1 - p, k)
            base = i * RB
            pltpu.sync_copy(acc_sh.at[pl.ds(base, RB), :], cch)
            bsrc = dsb_sh if k == 3 else dib_sh
            pltpu.sync_copy(bsrc.at[pl.ds(base, RB)], dibv)
            if k != 1:
                a_wait(i, p, k)

            def rowfn(r, carry2):
                bv = plsc.load_gather(dibv, [jnp.full((16,), r, jnp.int32)])
                for h in (0, 16):
                    cvv = cch[r, pl.ds(h, 16)]
                    if k == 1:
                        stg[p, r, pl.ds(h, 16)] = bv * cvv
                    elif k == 2:
                        stg[p, r, pl.ds(h, 16)] = bv * cvv - aux[p, r, pl.ds(h, 16)]
                    else:
                        stg[p, r, pl.ds(h, 16)] = (aux[p, r, pl.ds(h, 16)]
                                                   + bv * cvv) * 0.25
                return carry2
            lax.fori_loop(0, RB, rowfn, 0)
            u_fire(i, p)
            return carry
        with ns("pC_rescale"):
            lax.fori_loop(rb0, rb1, nblkfn, 0)
        u_wait(rb1 - 2, (rb1 - 2 - rb0) & 1)
        u_wait(rb1 - 1, (rb1 - 1 - rb0) & 1)

    for k in (1, 2, 3):
        layer_scatter()
        plsc.subcore_barrier()
        phase_c(k)
        plsc.subcore_barrier()

    def bchunk(j, carry):
        boff = s * BPT + j * 128
        pltpu.sync_copy(users.at[pl.ds(boff, 128)], pidx.at[0])
        pltpu.sync_copy(items.at[pl.ds(boff, 128)], pidx.at[1])
        for g in range(8):
            pidx[0, pl.ds(g * 16, 16)] = pidx[0, pl.ds(g * 16, 16)] + coff
            pidx[1, pl.ds(g * 16, 16)] = pidx[1, pl.ds(g * 16, 16)] + (coff + N_USERS)
        pltpu.async_copy(ubuf.at[pidx.at[0]], rows.at[0], gsem.at[0]).wait()
        pltpu.async_copy(ubuf.at[pidx.at[1]], rows.at[1], gsem.at[1]).wait()
        z16 = jnp.zeros((16,), jnp.int32)
        o16 = jnp.full((16,), 1, jnp.int32)
        for g in range(8):
            riv = g * 16 + iota16
            acc = jnp.zeros((16,), jnp.float32)
            for col in range(32):
                cv = jnp.full((16,), col, jnp.int32)
                acc = acc + (plsc.load_gather(rows, [z16, riv, cv])
                             * plsc.load_gather(rows, [o16, riv, cv]))
            gout[pl.ds(g * 16, 16)] = acc
        pltpu.sync_copy(gout, partials.at[pl.ds(c * BATCH + boff, 128)])
        return carry
    with ns("p5_gamma"):
        lax.fori_loop(0, 8, bchunk, 0)


_mesh = plsc.VectorSubcoreMesh(core_axis_name="c", subcore_axis_name="s",
                               num_cores=NC, num_subcores=NS)

_f32 = jnp.float32
_sc_call = functools.partial(
    pl.kernel,
    out_type=(
        jax.ShapeDtypeStruct((NC * BATCH,), _f32),
        jax.ShapeDtypeStruct((NC * NPAD, DH), _f32),
    ),
    mesh=_mesh,
    compiler_params=pltpu.CompilerParams(needs_layout_passes=False,
                                         use_tc_tiling_on_sc=False),
    scratch_types=[
        pltpu.VMEM_SHARED((NPAD, DH), _f32),
        pltpu.VMEM_SHARED((NPAD,), _f32),
        pltpu.VMEM_SHARED((NPAD,), _f32),
        pltpu.VMEM((8, 128), jnp.int32),
        pltpu.VMEM((3, 128, DH), _f32),
        pltpu.VMEM((128,), _f32),
        pltpu.VMEM((128,), _f32),
        pltpu.VMEM((RB,), _f32),
        pltpu.VMEM((RB,), _f32),
        pltpu.VMEM((RB,), _f32),
        pltpu.VMEM((2, RB, DH), _f32),
        pltpu.VMEM((2, RB, DH), _f32),
        pltpu.VMEM((RB, DH), _f32),
        pltpu.VMEM((128,), _f32),
        pltpu.SemaphoreType.DMA((4,)),
        pltpu.SemaphoreType.DMA((3,)),
        pltpu.SemaphoreType.DMA((2,)),
        pltpu.SemaphoreType.DMA((2,)),
        pltpu.SemaphoreType.DMA((2,)),
    ],
)(_body)


def kernel(users, items, user_emb, item_emb, edge_src, edge_dst):
    all_emb = jnp.concatenate([user_emb, item_emb], axis=0)
    e0p = jnp.pad(all_emb, ((0, NPAD - NN), (0, 0)))
    e0s = e0p.reshape(NPAD, NC, DH).transpose(1, 0, 2).reshape(NC * NPAD, DH)
    srcb = jnp.stack([edge_src, edge_src + NPAD]).reshape(NC, EBLK, 1, 128)
    dstb = jnp.broadcast_to(edge_dst.reshape(1, EBLK, 1, 128), (NC, EBLK, 1, 128))
    packed = jnp.concatenate([srcb, dstb], axis=2).reshape(NC * EBLK, 2, 128)
    partials = _sc_call(users, items, e0s, packed)[0]
    return partials[:BATCH] + partials[BATCH:]


# speedup vs baseline: 1.0647x; 1.0647x over previous
import functools

import jax
import jax.numpy as jnp
from jax import lax
from jax.experimental import pallas as pl
from jax.experimental.pallas import tpu as pltpu
from jax.experimental.pallas import tpu_sc as plsc

N_USERS = 20000
N_ITEMS = 30000
NN = N_USERS + N_ITEMS
NPAD = 50048
E = 800000
DH = 32
BATCH = 16384
NC, NS = 2, 16
EBLK = E // 128
RB = 64
RBLK = NPAD // RB
DBLK = NPAD // 128
BPT = BATCH // NS

_MAGIC = 0x5F3759DF


def _vrsqrt(x):
    i = lax.bitcast_convert_type(x, jnp.int32)
    y = lax.bitcast_convert_type(jnp.int32(_MAGIC) - (i >> 1), jnp.float32)
    for _ in range(3):
        y = y * (1.5 - 0.5 * x * y * y)
    return y


def _body(users, items, e0s, esrc2, edst,
          partials, ubuf,
          acc_sh, dib_sh, dsb_sh,
          sidx, didx, rows, z1, ones1, dchunk, dibv, dsbv,
          stg, aux, cch, gout,
          isem, gsem, ssem, uwsem, esem):
    c = lax.axis_index("c")
    s = lax.axis_index("s")
    coff = c * NPAD
    ceoff = c * E
    eb0 = (EBLK * s) // NS
    eb1 = (EBLK * (s + 1)) // NS
    rb0 = (RBLK * s) // NS
    rb1 = (RBLK * (s + 1)) // NS
    db0 = (DBLK * s) // NS
    db1 = (DBLK * (s + 1)) // NS
    iota16 = lax.iota(jnp.int32, 16)
    zv = jnp.zeros((16,), jnp.float32)
    ov = jnp.ones((16,), jnp.float32)
    ns = jax.named_scope

    for g in range(8):
        z1[pl.ds(g * 16, 16)] = zv
        ones1[pl.ds(g * 16, 16)] = ov
    for r in range(RB):
        stg[0, r, pl.ds(0, 16)] = zv
        stg[0, r, pl.ds(16, 16)] = zv

    def za_blk(i, carry):
        pltpu.sync_copy(stg.at[0], acc_sh.at[pl.ds(i * RB, RB), :])
        return carry

    def zd_blk(i, carry):
        pltpu.sync_copy(z1, dib_sh.at[pl.ds(i * 128, 128)])
        return carry
    with ns("p1_zero"):
        lax.fori_loop(rb0, rb1, za_blk, 0)
        lax.fori_loop(db0, db1, zd_blk, 0)
    plsc.subcore_barrier()

    def dfire(i):
        pltpu.async_copy(edst.at[pl.ds(i * 128, 128)], didx.at[i & 3], isem.at[i & 3])

    def dwait(i):
        pltpu.make_async_copy(edst.at[pl.ds(i * 128, 128)], didx.at[i & 3],
                              isem.at[i & 3]).wait()

    def dscat_wait(i):
        pltpu.make_async_copy(ones1, dib_sh.at[didx.at[i & 3]], ssem.at[i & 1]).wait()

    dfire(eb0)

    @pl.when(eb0 + 1 < eb1)
    def _d1():
        dfire(eb0 + 1)

    def deg_blk(i, carry):
        @pl.when(i - 2 >= eb0)
        def _w():
            dscat_wait(i - 2)

        @pl.when(i + 2 < eb1)
        def _f():
            dfire(i + 2)
        dwait(i)
        pltpu.async_copy(ones1, dib_sh.at[didx.at[i & 3]], ssem.at[i & 1], add=True)
        return carry
    with ns("p2_deg"):
        lax.fori_loop(eb0, eb1, deg_blk, 0)

    @pl.when(eb1 - 2 >= eb0)
    def _dw2():
        dscat_wait(eb1 - 2)
    dscat_wait(eb1 - 1)
    plsc.subcore_barrier()

    def e0_fire(i, slot):
        pltpu.async_copy(e0s.at[pl.ds(coff + i * RB, RB), :], aux.at[slot],
                         esem.at[slot])

    def e0_wait(i, slot):
        pltpu.make_async_copy(e0s.at[pl.ds(coff + i * RB, RB), :], aux.at[slot],
                              esem.at[slot]).wait()

    def u_fire(i, slot):
        pltpu.async_copy(stg.at[slot], ubuf.at[pl.ds(coff + i * RB, RB), :],
                         uwsem.at[slot])

    def u_wait(i, slot):
        pltpu.make_async_copy(stg.at[slot], ubuf.at[pl.ds(coff + i * RB, RB), :],
                              uwsem.at[slot]).wait()

    e0_fire(rb0, 0)

    def prep_blk(i, carry):
        p = (i - rb0) & 1

        @pl.when(i - 2 >= rb0)
        def _uw():
            u_wait(i - 2, p)

        @pl.when(i + 1 < rb1)
        def _ef():
            e0_fire(i + 1, 1 - p)
        base = i * RB
        pltpu.sync_copy(dib_sh.at[pl.ds(base, RB)], dchunk)
        for g in range(RB // 16):
            dv = dchunk[pl.ds(g * 16, 16)] + 1.0
            dibv[pl.ds(g * 16, 16)] = 1.0 / dv
            dsbv[pl.ds(g * 16, 16)] = _vrsqrt(dv)
        pltpu.sync_copy(dibv, dib_sh.at[pl.ds(base, RB)])
        pltpu.sync_copy(dsbv, dsb_sh.at[pl.ds(base, RB)])
        e0_wait(i, p)

        def rowfn(r2, carry2):
            for rr in (0, 1):
                r = 2 * r2 + rr
                bv = plsc.load_gather(dsbv, [jnp.full((16,), r, jnp.int32)])
                for h in (0, 16):
                    stg[p, r, pl.ds(h, 16)] = bv * aux[p, r, pl.ds(h, 16)]
            return carry2
        lax.fori_loop(0, RB // 2, rowfn, 0)
        u_fire(i, p)
        return carry
    with ns("p3_prep"):
        lax.fori_loop(rb0, rb1, prep_blk, 0)
    u_wait(rb1 - 2, (rb1 - 2 - rb0) & 1)
    u_wait(rb1 - 1, (rb1 - 1 - rb0) & 1)
    plsc.subcore_barrier()

    def idx_fire(i):
        q = i & 3
        pltpu.async_copy(esrc2.at[pl.ds(ceoff + i * 128, 128)], sidx.at[q],
                         isem.at[q])
        pltpu.async_copy(edst.at[pl.ds(i * 128, 128)], didx.at[q], isem.at[q])

    def idx_wait(i):
        q = i & 3
        pltpu.make_async_copy(esrc2.at[pl.ds(ceoff + i * 128, 128)], sidx.at[q],
                              isem.at[q]).wait()
        pltpu.make_async_copy(edst.at[pl.ds(i * 128, 128)], didx.at[q],
                              isem.at[q]).wait()

    def gather_fire(i):
        pltpu.async_copy(ubuf.at[sidx.at[i & 3]], rows.at[i % 3], gsem.at[i % 3])

    def gather_wait(i):
        pltpu.make_async_copy(ubuf.at[sidx.at[i & 3]], rows.at[i % 3],
                              gsem.at[i % 3]).wait()

    def ascat_wait(i):
        pltpu.make_async_copy(rows.at[i % 3], acc_sh.at[didx.at[i & 3]],
                              ssem.at[i & 1]).wait()

    def layer_scatter():
        idx_fire(eb0)
        idx_fire(eb0 + 1)
        idx_fire(eb0 + 2)
        idx_wait(eb0)
        gather_fire(eb0)
        idx_wait(eb0 + 1)
        gather_fire(eb0 + 1)

        def eblk(i, carry):
            @pl.when(i - 1 >= eb0)
            def _sw():
                ascat_wait(i - 1)

            @pl.when(i + 3 < eb1)
            def _if():
                idx_fire(i + 3)

            @pl.when(i + 2 < eb1)
            def _gf():
                idx_wait(i + 2)
                gather_fire(i + 2)
            gather_wait(i)
            pltpu.async_copy(rows.at[i % 3], acc_sh.at[didx.at[i & 3]],
                             ssem.at[i & 1], add=True)
            return carry
        with ns("pB_scatter"):
            lax.fori_loop(eb0, eb1, eblk, 0)
        ascat_wait(eb1 - 1)

    def a_fire(i, slot, k):
        if k == 2:
            pltpu.async_copy(ubuf.at[pl.ds(coff + i * RB, RB), :], aux.at[slot],
                             esem.at[slot])
        else:
            e0_fire(i, slot)

    def a_wait(i, slot, k):
        if k == 2:
            pltpu.make_async_copy(ubuf.at[pl.ds(coff + i * RB, RB), :],
                                  aux.at[slot], esem.at[slot]).wait()
        else:
            e0_wait(i, slot)

    def phase_c(k):
        if k != 1:
            a_fire(rb0, 0, k)

        def nblkfn(i, carry):
            p = (i - rb0) & 1

            @pl.when(i - 2 >= rb0)
            def _uw():
                u_wait(i - 2, p)

            if k != 1:
                @pl.when(i + 1 < rb1)
                def _af():
                    a_fire(i + 1, 1 - p, k)
            base = i * RB
            pltpu.sync_copy(acc_sh.at[pl.ds(base, RB), :], cch)
            bsrc = dsb_sh if k == 3 else dib_sh
            pltpu.sync_copy(bsrc.at[pl.ds(base, RB)], dibv)
            if k != 1:
                a_wait(i, p, k)

            def rowfn(r2, carry2):
                for rr in (0, 1):
                    r = 2 * r2 + rr
                    bv = plsc.load_gather(dibv, [jnp.full((16,), r, jnp.int32)])
                    for h in (0, 16):
                        cvv = cch[r, pl.ds(h, 16)]
                        if k == 1:
                            stg[p, r, pl.ds(h, 16)] = bv * cvv
                        elif k == 2:
                            stg[p, r, pl.ds(h, 16)] = (bv * cvv
                                                       - aux[p, r, pl.ds(h, 16)])
                        else:
                            stg[p, r, pl.ds(h, 16)] = (aux[p, r, pl.ds(h, 16)]
                                                       + bv * cvv) * 0.25
                return carry2
            lax.fori_loop(0, RB // 2, rowfn, 0)
            u_fire(i, p)
            return carry
        with ns("pC_rescale"):
            lax.fori_loop(rb0, rb1, nblkfn, 0)
        u_wait(rb1 - 2, (rb1 - 2 - rb0) & 1)
        u_wait(rb1 - 1, (rb1 - 1 - rb0) & 1)

    for k in (1, 2, 3):
        layer_scatter()
        plsc.subcore_barrier()
        phase_c(k)
        plsc.subcore_barrier()

    def bidx_fire(j):
        q = j & 1
        boff = s * BPT + j * 128
        pltpu.async_copy(users.at[pl.ds(boff, 128)], sidx.at[q], isem.at[q])
        pltpu.async_copy(items.at[pl.ds(boff, 128)], didx.at[q], isem.at[q])

    def bidx_wait(j):
        q = j & 1
        boff = s * BPT + j * 128
        pltpu.make_async_copy(users.at[pl.ds(boff, 128)], sidx.at[q],
                              isem.at[q]).wait()
        pltpu.make_async_copy(items.at[pl.ds(boff, 128)], didx.at[q],
                              isem.at[q]).wait()

    bidx_fire(0)

    def bchunk(j, carry):
        q = j & 1

        @pl.when(j + 1 < 8)
        def _bf():
            bidx_fire(j + 1)
        bidx_wait(j)
        for g in range(8):
            sidx[q, pl.ds(g * 16, 16)] = sidx[q, pl.ds(g * 16, 16)] + coff
            didx[q, pl.ds(g * 16, 16)] = didx[q, pl.ds(g * 16, 16)] + (coff + N_USERS)
        pltpu.async_copy(ubuf.at[sidx.at[q]], rows.at[0], gsem.at[0])
        pltpu.async_copy(ubuf.at[didx.at[q]], rows.at[1], gsem.at[1])
        pltpu.make_async_copy(ubuf.at[sidx.at[q]], rows.at[0], gsem.at[0]).wait()
        pltpu.make_async_copy(ubuf.at[didx.at[q]], rows.at[1], gsem.at[1]).wait()
        z16 = jnp.zeros((16,), jnp.int32)
        o16 = jnp.full((16,), 1, jnp.int32)
        boff = s * BPT + j * 128
        for g in range(8):
            riv = g * 16 + iota16
            acc = jnp.zeros((16,), jnp.float32)
            for col in range(32):
                cv = jnp.full((16,), col, jnp.int32)
                acc = acc + (plsc.load_gather(rows, [z16, riv, cv])
                             * plsc.load_gather(rows, [o16, riv, cv]))
            gout[pl.ds(g * 16, 16)] = acc
        pltpu.sync_copy(gout, partials.at[pl.ds(c * BATCH + boff, 128)])
        return carry
    with ns("p5_gamma"):
        lax.fori_loop(0, 8, bchunk, 0)


_mesh = plsc.VectorSubcoreMesh(core_axis_name="c", subcore_axis_name="s",
                               num_cores=NC, num_subcores=NS)

_f32 = jnp.float32
_sc_call = functools.partial(
    pl.kernel,
    out_type=(
        jax.ShapeDtypeStruct((NC * BATCH,), _f32),
        jax.ShapeDtypeStruct((NC * NPAD, DH), _f32),
    ),
    mesh=_mesh,
    compiler_params=pltpu.CompilerParams(needs_layout_passes=False,
                                         use_tc_tiling_on_sc=False),
    scratch_types=[
        pltpu.VMEM_SHARED((NPAD, DH), _f32),
        pltpu.VMEM_SHARED((NPAD,), _f32),
        pltpu.VMEM_SHARED((NPAD,), _f32),
        pltpu.VMEM((4, 128), jnp.int32),
        pltpu.VMEM((4, 128), jnp.int32),
        pltpu.VMEM((3, 128, DH), _f32),
        pltpu.VMEM((128,), _f32),
        pltpu.VMEM((128,), _f32),
        pltpu.VMEM((RB,), _f32),
        pltpu.VMEM((RB,), _f32),
        pltpu.VMEM((RB,), _f32),
        pltpu.VMEM((2, RB, DH), _f32),
        pltpu.VMEM((2, RB, DH), _f32),
        pltpu.VMEM((RB, DH), _f32),
        pltpu.VMEM((128,), _f32),
        pltpu.SemaphoreType.DMA((4,)),
        pltpu.SemaphoreType.DMA((3,)),
        pltpu.SemaphoreType.DMA((2,)),
        pltpu.SemaphoreType.DMA((2,)),
        pltpu.SemaphoreType.DMA((2,)),
    ],
)(_body)


def kernel(users, items, user_emb, item_emb, edge_src, edge_dst):
    all_emb = jnp.concatenate([user_emb, item_emb], axis=0)
    e0p = jnp.pad(all_emb, ((0, NPAD - NN), (0, 0)))
    e0s = e0p.reshape(NPAD, NC, DH).transpose(1, 0, 2).reshape(NC * NPAD, DH)
    esrc2 = jnp.concatenate([edge_src, edge_src + NPAD])
    partials = _sc_call(users, items, e0s, esrc2, edge_dst)[0]
    return partials[:BATCH] + partials[BATCH:]


# speedup vs baseline: 1.0893x; 1.0231x over previous
import functools

import jax
import jax.numpy as jnp
from jax import lax
from jax.experimental import pallas as pl
from jax.experimental.pallas import tpu as pltpu
from jax.experimental.pallas import tpu_sc as plsc

N_USERS = 20000
N_ITEMS = 30000
NN = N_USERS + N_ITEMS
NPAD = 50048
E = 800000
DH = 32
BATCH = 16384
NC, NS = 2, 16
EBLK = E // 256
RB = 32
RBLK = NPAD // RB
DBLK = NPAD // 128
BPT = BATCH // NS

_MAGIC = 0x5F3759DF


def _vrsqrt(x):
    i = lax.bitcast_convert_type(x, jnp.int32)
    y = lax.bitcast_convert_type(jnp.int32(_MAGIC) - (i >> 1), jnp.float32)
    for _ in range(3):
        y = y * (1.5 - 0.5 * x * y * y)
    return y


def _body(users, items, e0s, esrc3, edst3,
          partials, ubuf,
          acc_sh, dib_sh, dsb_sh,
          sidx, didx, rows, z1, ones1, dchunk, dibv, dsbv,
          stg, aux, cch, gout,
          isem, gsem, ssem, uwsem, esem):
    c = lax.axis_index("c")
    s = lax.axis_index("s")
    coff = c * NPAD
    ceb = c * EBLK
    eb0 = (EBLK * s) // NS
    eb1 = (EBLK * (s + 1)) // NS
    rb0 = (RBLK * s) // NS
    rb1 = (RBLK * (s + 1)) // NS
    db0 = (DBLK * s) // NS
    db1 = (DBLK * (s + 1)) // NS
    iota16 = lax.iota(jnp.int32, 16)
    zv = jnp.zeros((16,), jnp.float32)
    ov = jnp.ones((16,), jnp.float32)
    ns = jax.named_scope

    for g in range(8):
        z1[pl.ds(g * 16, 16)] = zv
    for g in range(16):
        ones1[pl.ds(g * 16, 16)] = ov
    for r in range(RB):
        stg[0, r, pl.ds(0, 16)] = zv
        stg[0, r, pl.ds(16, 16)] = zv

    def za_blk(i, carry):
        pltpu.sync_copy(stg.at[0], acc_sh.at[pl.ds(i * RB, RB), :])
        return carry

    def zd_blk(i, carry):
        pltpu.sync_copy(z1, dib_sh.at[pl.ds(i * 128, 128)])
        return carry
    with ns("p1_zero"):
        lax.fori_loop(rb0, rb1, za_blk, 0)
        lax.fori_loop(db0, db1, zd_blk, 0)
    plsc.subcore_barrier()

    def dfire(i):
        q = lax.rem(i, 3)
        pltpu.async_copy(edst3.at[i], didx.at[q], isem.at[q])

    def dwait(i):
        q = lax.rem(i, 3)
        pltpu.make_async_copy(edst3.at[i], didx.at[q], isem.at[q]).wait()

    def dscat_wait(i):
        pltpu.make_async_copy(ones1, dib_sh.at[didx.at[lax.rem(i, 3), 0]],
                              ssem.at[i & 1]).wait()

    dfire(eb0)

    @pl.when(eb0 + 1 < eb1)
    def _d1():
        dfire(eb0 + 1)

    def deg_blk(i, carry):
        @pl.when(i - 2 >= eb0)
        def _w():
            dscat_wait(i - 2)

        @pl.when(i + 2 < eb1)
        def _f():
            dfire(i + 2)
        dwait(i)
        pltpu.async_copy(ones1, dib_sh.at[didx.at[lax.rem(i, 3), 0]], ssem.at[i & 1], add=True)
        return carry
    with ns("p2_deg"):
        lax.fori_loop(eb0, eb1, deg_blk, 0)

    @pl.when(eb1 - 2 >= eb0)
    def _dw2():
        dscat_wait(eb1 - 2)
    dscat_wait(eb1 - 1)
    plsc.subcore_barrier()

    def e0_fire(i, slot):
        pltpu.async_copy(e0s.at[pl.ds(coff + i * RB, RB), :], aux.at[slot],
                         esem.at[slot])

    def e0_wait(i, slot):
        pltpu.make_async_copy(e0s.at[pl.ds(coff + i * RB, RB), :], aux.at[slot],
                              esem.at[slot]).wait()

    def u_fire(i, slot):
        pltpu.async_copy(stg.at[slot], ubuf.at[pl.ds(coff + i * RB, RB), :],
                         uwsem.at[slot])

    def u_wait(i, slot):
        pltpu.make_async_copy(stg.at[slot], ubuf.at[pl.ds(coff + i * RB, RB), :],
                              uwsem.at[slot]).wait()

    e0_fire(rb0, 0)

    def prep_blk(i, carry):
        p = (i - rb0) & 1

        @pl.when(i - 2 >= rb0)
        def _uw():
            u_wait(i - 2, p)

        @pl.when(i + 1 < rb1)
        def _ef():
            e0_fire(i + 1, 1 - p)
        base = i * RB
        pltpu.sync_copy(dib_sh.at[pl.ds(base, RB)], dchunk)
        for g in range(RB // 16):
            dv = dchunk[pl.ds(g * 16, 16)] + 1.0
            dibv[pl.ds(g * 16, 16)] = 1.0 / dv
            dsbv[pl.ds(g * 16, 16)] = _vrsqrt(dv)
        pltpu.sync_copy(dibv, dib_sh.at[pl.ds(base, RB)])
        pltpu.sync_copy(dsbv, dsb_sh.at[pl.ds(base, RB)])
        e0_wait(i, p)

        def rowfn(r2, carry2):
            for rr in (0, 1):
                r = 2 * r2 + rr
                bv = plsc.load_gather(dsbv, [jnp.full((16,), r, jnp.int32)])
                for h in (0, 16):
                    stg[p, r, pl.ds(h, 16)] = bv * aux[p, r, pl.ds(h, 16)]
            return carry2
        lax.fori_loop(0, RB // 2, rowfn, 0)
        u_fire(i, p)
        return carry
    with ns("p3_prep"):
        lax.fori_loop(rb0, rb1, prep_blk, 0)
    u_wait(rb1 - 2, (rb1 - 2 - rb0) & 1)
    u_wait(rb1 - 1, (rb1 - 1 - rb0) & 1)
    plsc.subcore_barrier()

    def idx_fire(i):
        q = lax.rem(i, 3)
        pltpu.async_copy(esrc3.at[ceb + i], sidx.at[q], isem.at[q])
        pltpu.async_copy(edst3.at[i], didx.at[q], isem.at[q])

    def idx_wait(i):
        q = lax.rem(i, 3)
        pltpu.make_async_copy(esrc3.at[ceb + i], sidx.at[q], isem.at[q]).wait()
        pltpu.make_async_copy(edst3.at[i], didx.at[q], isem.at[q]).wait()

    def gather_fire(i):
        q = lax.rem(i, 3)
        pltpu.async_copy(ubuf.at[sidx.at[q, 0]], rows.at[i & 1], gsem.at[i & 1])

    def gather_wait(i):
        q = lax.rem(i, 3)
        pltpu.make_async_copy(ubuf.at[sidx.at[q, 0]], rows.at[i & 1],
                              gsem.at[i & 1]).wait()

    def ascat_wait(i):
        q = lax.rem(i, 3)
        pltpu.make_async_copy(rows.at[i & 1], acc_sh.at[didx.at[q, 0]],
                              ssem.at[i & 1]).wait()

    def layer_scatter():
        idx_fire(eb0)
        idx_fire(eb0 + 1)
        idx_wait(eb0)
        gather_fire(eb0)

        def eblk(i, carry):
            @pl.when(i - 1 >= eb0)
            def _sw():
                ascat_wait(i - 1)

            @pl.when(i + 2 < eb1)
            def _if():
                idx_fire(i + 2)

            @pl.when(i + 1 < eb1)
            def _gf():
                idx_wait(i + 1)
                gather_fire(i + 1)
            gather_wait(i)
            pltpu.async_copy(rows.at[i & 1], acc_sh.at[didx.at[lax.rem(i, 3), 0]],
                             ssem.at[i & 1], add=True)
            return carry
        with ns("pB_scatter"):
            lax.fori_loop(eb0, eb1, eblk, 0)
        ascat_wait(eb1 - 1)

    def a_fire(i, slot, k):
        if k == 2:
            pltpu.async_copy(ubuf.at[pl.ds(coff + i * RB, RB), :], aux.at[slot],
                             esem.at[slot])
        else:
            e0_fire(i, slot)

    def a_wait(i, slot, k):
        if k == 2:
            pltpu.make_async_copy(ubuf.at[pl.ds(coff + i * RB, RB), :],
                                  aux.at[slot], esem.at[slot]).wait()
        else:
            e0_wait(i, slot)

    def phase_c(k):
        if k != 1:
            a_fire(rb0, 0, k)

        def nblkfn(i, carry):
            p = (i - rb0) & 1

            @pl.when(i - 2 >= rb0)
            def _uw():
                u_wait(i - 2, p)

            if k != 1:
                @pl.when(i + 1 < rb1)
                def _af():
                    a_fire(i + 1, 1 - p, k)
            base = i * RB
            pltpu.sync_copy(acc_sh.at[pl.ds(base, RB), :], cch)
            bsrc = dsb_sh if k == 3 else dib_sh
            pltpu.sync_copy(bsrc.at[pl.ds(base, RB)], dibv)
            if k != 1:
                a_wait(i, p, k)

            def rowfn(r2, carry2):
                for rr in (0, 1):
                    r = 2 * r2 + rr
                    bv = plsc.load_gather(dibv, [jnp.full((16,), r, jnp.int32)])
                    for h in (0, 16):
                        cvv = cch[r, pl.ds(h, 16)]
                        if k == 1:
                            stg[p, r, pl.ds(h, 16)] = bv * cvv
                        elif k == 2:
                            stg[p, r, pl.ds(h, 16)] = (bv * cvv
                                                       - aux[p, r, pl.ds(h, 16)])
                        else:
                            stg[p, r, pl.ds(h, 16)] = (aux[p, r, pl.ds(h, 16)]
                                                       + bv * cvv) * 0.25
                return carry2
            lax.fori_loop(0, RB // 2, rowfn, 0)
            u_fire(i, p)
            return carry
        with ns("pC_rescale"):
            lax.fori_loop(rb0, rb1, nblkfn, 0)
        u_wait(rb1 - 2, (rb1 - 2 - rb0) & 1)
        u_wait(rb1 - 1, (rb1 - 1 - rb0) & 1)

    for k in (1, 2, 3):
        layer_scatter()
        plsc.subcore_barrier()
        phase_c(k)
        plsc.subcore_barrier()

    def bidx_fire(j):
        q = j & 1
        boff = s * BPT + j * 128
        pltpu.async_copy(users.at[pl.ds(boff, 128)], sidx.at[q, 0, pl.ds(0, 128)],
                         isem.at[q])
        pltpu.async_copy(items.at[pl.ds(boff, 128)], didx.at[q, 0, pl.ds(0, 128)],
                         isem.at[q])

    def bidx_wait(j):
        q = j & 1
        boff = s * BPT + j * 128
        pltpu.make_async_copy(users.at[pl.ds(boff, 128)],
                              sidx.at[q, 0, pl.ds(0, 128)], isem.at[q]).wait()
        pltpu.make_async_copy(items.at[pl.ds(boff, 128)],
                              didx.at[q, 0, pl.ds(0, 128)], isem.at[q]).wait()

    bidx_fire(0)

    def bchunk(j, carry):
        q = j & 1

        @pl.when(j + 1 < 8)
        def _bf():
            bidx_fire(j + 1)
        bidx_wait(j)
        for g in range(8):
            sidx[q, 0, pl.ds(g * 16, 16)] = sidx[q, 0, pl.ds(g * 16, 16)] + coff
            didx[q, 0, pl.ds(g * 16, 16)] = (didx[q, 0, pl.ds(g * 16, 16)]
                                             + (coff + N_USERS))
        u_idx = sidx.at[q, 0, pl.ds(0, 128)]
        i_idx = didx.at[q, 0, pl.ds(0, 128)]
        pltpu.async_copy(ubuf.at[u_idx], rows.at[0, pl.ds(0, 128)], gsem.at[0])
        pltpu.async_copy(ubuf.at[i_idx], rows.at[1, pl.ds(0, 128)], gsem.at[1])
        pltpu.make_async_copy(ubuf.at[u_idx], rows.at[0, pl.ds(0, 128)],
                              gsem.at[0]).wait()
        pltpu.make_async_copy(ubuf.at[i_idx], rows.at[1, pl.ds(0, 128)],
                              gsem.at[1]).wait()
        z16 = jnp.zeros((16,), jnp.int32)
        o16 = jnp.full((16,), 1, jnp.int32)
        boff = s * BPT + j * 128
        for g in range(8):
            riv = g * 16 + iota16
            acc = jnp.zeros((16,), jnp.float32)
            for col in range(32):
                cv = jnp.full((16,), col, jnp.int32)
                acc = acc + (plsc.load_gather(rows, [z16, riv, cv])
                             * plsc.load_gather(rows, [o16, riv, cv]))
            gout[pl.ds(g * 16, 16)] = acc
        pltpu.sync_copy(gout, partials.at[pl.ds(c * BATCH + boff, 128)])
        return carry
    with ns("p5_gamma"):
        lax.fori_loop(0, 8, bchunk, 0)


_mesh = plsc.VectorSubcoreMesh(core_axis_name="c", subcore_axis_name="s",
                               num_cores=NC, num_subcores=NS)

_f32 = jnp.float32
_sc_call = functools.partial(
    pl.kernel,
    out_type=(
        jax.ShapeDtypeStruct((NC * BATCH,), _f32),
        jax.ShapeDtypeStruct((NC * NPAD, DH), _f32),
    ),
    mesh=_mesh,
    compiler_params=pltpu.CompilerParams(needs_layout_passes=False,
                                         use_tc_tiling_on_sc=False),
    scratch_types=[
        pltpu.VMEM_SHARED((NPAD, DH), _f32),
        pltpu.VMEM_SHARED((NPAD,), _f32),
        pltpu.VMEM_SHARED((NPAD,), _f32),
        pltpu.VMEM((3, 1, 256), jnp.int32),
        pltpu.VMEM((3, 1, 256), jnp.int32),
        pltpu.VMEM((2, 256, DH), _f32),
        pltpu.VMEM((128,), _f32),
        pltpu.VMEM((256,), _f32),
        pltpu.VMEM((RB,), _f32),
        pltpu.VMEM((RB,), _f32),
        pltpu.VMEM((RB,), _f32),
        pltpu.VMEM((2, RB, DH), _f32),
        pltpu.VMEM((2, RB, DH), _f32),
        pltpu.VMEM((RB, DH), _f32),
        pltpu.VMEM((128,), _f32),
        pltpu.SemaphoreType.DMA((4,)),
        pltpu.SemaphoreType.DMA((3,)),
        pltpu.SemaphoreType.DMA((2,)),
        pltpu.SemaphoreType.DMA((2,)),
        pltpu.SemaphoreType.DMA((2,)),
    ],
)(_body)


def kernel(users, items, user_emb, item_emb, edge_src, edge_dst):
    all_emb = jnp.concatenate([user_emb, item_emb], axis=0)
    e0p = jnp.pad(all_emb, ((0, NPAD - NN), (0, 0)))
    e0s = e0p.reshape(NPAD, NC, DH).transpose(1, 0, 2).reshape(NC * NPAD, DH)
    esrc3 = jnp.stack([edge_src, edge_src + NPAD]).reshape(NC * EBLK, 1, 256)
    edst3 = edge_dst.reshape(EBLK, 1, 256)
    partials = _sc_call(users, items, e0s, esrc3, edst3)[0]
    return partials[:BATCH] + partials[BATCH:]
